# baseline (speedup 1.0000x reference)
import functools

import jax
import jax.numpy as jnp
from jax.experimental import pallas as pl
from jax.experimental.pallas import tpu as pltpu
from jax.experimental.pallas import tpu_sc as plsc

_PAIRS = [(2, 15), (1, 204), (2, 8812), (1, 7), (2, 15), (1, 56013),
          (2, 77105), (1, 204), (2, 3), (1, 99998), (2, 45000), (1, 12345),
          (2, 8812), (1, 7), (2, 67890), (1, 23456), (2, 15), (1, 88001),
          (2, 500), (1, 204), (2, 77105), (1, 4096), (2, 31415), (1, 27182),
          (2, 3), (1, 56013), (2, 99999), (1, 1), (2, 500), (1, 12345),
          (2, 8812), (1, 65536), (2, 42), (1, 7), (2, 31415), (1, 99998),
          (2, 15), (1, 204), (2, 45000), (1, 88001)]

_M = len(_PAIRS)

_MESH = plsc.ScalarSubcoreMesh(axis_name="c", num_cores=1)


@functools.partial(
    pl.kernel,
    mesh=_MESH,
    out_type=jax.ShapeDtypeStruct((_M, 4, 128), jnp.float32),
    scratch_types=[
        pltpu.SemaphoreType.DMA,
    ],
)
def _gather_sc(t1_hbm, t2_hbm, out_hbm, sem):
    tables = {1: t1_hbm, 2: t2_hbm}
    copies = [pltpu.async_copy(tables[l].at[o], out_hbm.at[i], sem)
              for i, (l, o) in enumerate(_PAIRS)]
    for c in copies:
        c.wait()


def kernel(layer_1, layer_2):
    return _gather_sc(layer_1, layer_2)
